# baseline (speedup 1.0000x reference)
import functools

import jax
import jax.numpy as jnp
from jax import lax
from jax.experimental import pallas as pl
from jax.experimental.pallas import tpu as pltpu
from jax.experimental.pallas import tpu_sc as plsc

NC = 2
NS = 16
TILES = NC * NS
K = 128
BPT = 80
IB = 16
NB = BPT // IB


def _sc_partials(x, srcp, dstp, zrows, *, n_pad, d):
    stripe = n_pad // NS

    @functools.partial(
        pl.kernel,
        out_type=jax.ShapeDtypeStruct((NC, n_pad, d), jnp.float32),
        mesh=plsc.VectorSubcoreMesh(core_axis_name="c", subcore_axis_name="s"),
        scratch_types=[
            pltpu.VMEM((IB, K), jnp.int32),
            pltpu.VMEM((IB, K), jnp.int32),
            pltpu.VMEM((IB, K), jnp.int32),
            pltpu.VMEM((IB, K), jnp.int32),
            pltpu.VMEM((K, d), jnp.float32),
            pltpu.VMEM((K, d), jnp.float32),
            pltpu.VMEM_SHARED((n_pad, d), jnp.float32),
            pltpu.SemaphoreType.DMA,
            pltpu.SemaphoreType.DMA,
            pltpu.SemaphoreType.DMA,
            pltpu.SemaphoreType.DMA,
            pltpu.SemaphoreType.DMA,
            pltpu.SemaphoreType.DMA,
        ],
    )
    def sc_kernel(x_hbm, srcp_hbm, dstp_hbm, zrows_hbm, out_hbm,
                  sidxA, didxA, sidxB, didxB, rows0, rows1, acc,
                  gs0, gs1, ss0, ss1, isemA, isemB):
        c = lax.axis_index("c")
        s = lax.axis_index("s")
        t = c * NS + s

        idxbufs = [(sidxA, didxA, isemA), (sidxB, didxB, isemB)]

        def start_idx(b, bufs):
            sidx, didx, isem = bufs
            base = t * BPT + b * IB
            pltpu.async_copy(srcp_hbm.at[pl.ds(base, IB)], sidx, isem)
            pltpu.async_copy(dstp_hbm.at[pl.ds(base, IB)], didx, isem)

        def wait_idx(bufs):
            sidx, didx, isem = bufs
            pltpu.make_async_copy(srcp_hbm.at[pl.ds(0, IB)], sidx, isem).wait()
            pltpu.make_async_copy(dstp_hbm.at[pl.ds(0, IB)], didx, isem).wait()

        def start_g(sidx, j, rows, sem):
            pltpu.async_copy(x_hbm.at[sidx.at[j]], rows, sem)

        def wait_g(sidx, j, rows, sem):
            pltpu.make_async_copy(x_hbm.at[sidx.at[j]], rows, sem).wait()

        def start_scat(didx, j, rows, sem):
            pltpu.async_copy(rows, acc.at[didx.at[j]], sem, add=True)

        def wait_scat(didx, j, rows, sem):
            pltpu.make_async_copy(rows, acc.at[didx.at[j]], sem).wait()

        start_idx(0, idxbufs[0])
        wait_idx(idxbufs[0])
        start_idx(1, idxbufs[1])
        start_g(sidxA, 0, rows0, gs0)
        start_g(sidxA, 1, rows1, gs1)

        pltpu.sync_copy(zrows_hbm, acc.at[pl.ds(s * stripe, stripe)])
        plsc.subcore_barrier()

        for b in range(NB):
            sidx, didx, _ = idxbufs[b % 2]
            nxt = idxbufs[(b + 1) % 2]

            @pl.loop(0, IB, step=2)
            def _(j, sidx=sidx, didx=didx):
                wait_g(sidx, j, rows0, gs0)
                start_scat(didx, j, rows0, ss0)
                wait_g(sidx, j + 1, rows1, gs1)
                start_scat(didx, j + 1, rows1, ss1)

                @pl.when(j + 2 < IB)
                def _():
                    wait_scat(didx, j, rows0, ss0)
                    start_g(sidx, j + 2, rows0, gs0)
                    wait_scat(didx, j + 1, rows1, ss1)
                    start_g(sidx, j + 3, rows1, gs1)

            if b + 1 < NB:
                wait_idx(nxt)
                wait_scat(didx, IB - 2, rows0, ss0)
                start_g(nxt[0], 0, rows0, gs0)
                wait_scat(didx, IB - 1, rows1, ss1)
                start_g(nxt[0], 1, rows1, gs1)
            if b + 2 < NB:
                start_idx(b + 2, idxbufs[b % 2])

        wait_scat(didxA, IB - 2, rows0, ss0)
        wait_scat(didxA, IB - 1, rows1, ss1)

        plsc.subcore_barrier()

        pltpu.sync_copy(acc.at[pl.ds(s * stripe, stripe)],
                        out_hbm.at[c].at[pl.ds(s * stripe, stripe)])

    return sc_kernel(x, srcp, dstp, zrows)


def _tc_add_body(p_ref, o_ref):
    n = o_ref.shape[0]
    o_ref[...] = p_ref[0, :n, :] + p_ref[1, :n, :]


def kernel(x, edge_index):
    n, d = x.shape
    e = edge_index.shape[1]
    n_pad = ((n + NS * 8 - 1) // (NS * 8)) * (NS * 8)
    e_pad = TILES * BPT * K
    pad = e_pad - e

    dummy_dst = n + jnp.arange(pad, dtype=jnp.int32) % (n_pad - n)
    dummy_src = jnp.arange(pad, dtype=jnp.int32) % n
    src = jnp.concatenate(
        [edge_index[0], dummy_src]).reshape(-1, K)
    dst = jnp.concatenate(
        [edge_index[1], dummy_dst]).reshape(-1, K)
    zrows = jnp.zeros((n_pad // NS, d), jnp.float32)

    partial = _sc_partials(x, src, dst, zrows, n_pad=n_pad, d=d)

    return pl.pallas_call(
        _tc_add_body,
        out_shape=jax.ShapeDtypeStruct((n, d), jnp.float32),
    )(partial)


# speedup vs baseline: 1.2139x; 1.2139x over previous
import functools

import jax
import jax.numpy as jnp
from jax import lax
from jax.experimental import pallas as pl
from jax.experimental.pallas import tpu as pltpu
from jax.experimental.pallas import tpu_sc as plsc

NC = 2
NS = 16
TILES = NC * NS
K = 128
BPT = 80
IB = 16
NB = BPT // IB


def _sc_partials(x, srcp, dstp, zrows, *, n_pad, d):
    stripe = n_pad // NS

    @functools.partial(
        pl.kernel,
        out_type=jax.ShapeDtypeStruct((NC, n_pad, d), jnp.float32),
        mesh=plsc.VectorSubcoreMesh(core_axis_name="c", subcore_axis_name="s"),
        scratch_types=[
            pltpu.VMEM((IB, K), jnp.int32),
            pltpu.VMEM((IB, K), jnp.int32),
            pltpu.VMEM((IB, K), jnp.int32),
            pltpu.VMEM((IB, K), jnp.int32),
            pltpu.VMEM((K, d), jnp.float32),
            pltpu.VMEM((K, d), jnp.float32),
            pltpu.VMEM_SHARED((n_pad, d), jnp.float32),
            pltpu.SemaphoreType.DMA,
            pltpu.SemaphoreType.DMA,
            pltpu.SemaphoreType.DMA,
            pltpu.SemaphoreType.DMA,
            pltpu.SemaphoreType.DMA,
            pltpu.SemaphoreType.DMA,
        ],
    )
    def sc_kernel(x_hbm, srcp_hbm, dstp_hbm, zrows_hbm, out_hbm,
                  sidxA, didxA, sidxB, didxB, rows0, rows1, acc,
                  gs0, gs1, ss0, ss1, isemA, isemB):
        c = lax.axis_index("c")
        s = lax.axis_index("s")
        t = c * NS + s

        idxbufs = [(sidxA, didxA, isemA), (sidxB, didxB, isemB)]

        def start_idx(b, bufs):
            sidx, didx, isem = bufs
            base = t * BPT + b * IB
            pltpu.async_copy(srcp_hbm.at[pl.ds(base, IB)], sidx, isem)
            pltpu.async_copy(dstp_hbm.at[pl.ds(base, IB)], didx, isem)

        def wait_idx(bufs):
            sidx, didx, isem = bufs
            pltpu.make_async_copy(srcp_hbm.at[pl.ds(0, IB)], sidx, isem).wait()
            pltpu.make_async_copy(dstp_hbm.at[pl.ds(0, IB)], didx, isem).wait()

        def start_g(sidx, j, rows, sem):
            pltpu.async_copy(x_hbm.at[sidx.at[j]], rows, sem)

        def wait_g(sidx, j, rows, sem):
            pltpu.make_async_copy(x_hbm.at[sidx.at[j]], rows, sem).wait()

        def scat(didx, j, rows):
            pltpu.sync_copy(rows, acc.at[didx.at[j]], add=True)

        start_idx(0, idxbufs[0])
        wait_idx(idxbufs[0])
        start_idx(1, idxbufs[1])
        start_g(sidxA, 0, rows0, gs0)
        start_g(sidxA, 1, rows1, gs1)

        pltpu.sync_copy(zrows_hbm, acc.at[pl.ds(s * stripe, stripe)])
        plsc.subcore_barrier()

        for b in range(NB):
            sidx, didx, _ = idxbufs[b % 2]
            nxt = idxbufs[(b + 1) % 2]

            @pl.loop(0, IB, step=2)
            def _(j, sidx=sidx, didx=didx):
                wait_g(sidx, j, rows0, gs0)
                scat(didx, j, rows0)

                @pl.when(j + 2 < IB)
                def _():
                    start_g(sidx, j + 2, rows0, gs0)

                wait_g(sidx, j + 1, rows1, gs1)
                scat(didx, j + 1, rows1)

                @pl.when(j + 3 < IB)
                def _():
                    start_g(sidx, j + 3, rows1, gs1)

            if b + 1 < NB:
                wait_idx(nxt)
                start_g(nxt[0], 0, rows0, gs0)
                start_g(nxt[0], 1, rows1, gs1)
            if b + 2 < NB:
                start_idx(b + 2, idxbufs[b % 2])

        plsc.subcore_barrier()

        pltpu.sync_copy(acc.at[pl.ds(s * stripe, stripe)],
                        out_hbm.at[c].at[pl.ds(s * stripe, stripe)])

    return sc_kernel(x, srcp, dstp, zrows)


def _tc_add_body(p_ref, o_ref):
    n = o_ref.shape[0]
    o_ref[...] = p_ref[0, :n, :] + p_ref[1, :n, :]


def kernel(x, edge_index):
    n, d = x.shape
    e = edge_index.shape[1]
    n_pad = ((n + NS * 8 - 1) // (NS * 8)) * (NS * 8)
    e_pad = TILES * BPT * K
    pad = e_pad - e

    dummy_dst = n + jnp.arange(pad, dtype=jnp.int32) % (n_pad - n)
    dummy_src = jnp.arange(pad, dtype=jnp.int32) % n
    src = jnp.concatenate(
        [edge_index[0], dummy_src]).reshape(-1, K)
    dst = jnp.concatenate(
        [edge_index[1], dummy_dst]).reshape(-1, K)
    zrows = jnp.zeros((n_pad // NS, d), jnp.float32)

    partial = _sc_partials(x, src, dst, zrows, n_pad=n_pad, d=d)

    return pl.pallas_call(
        _tc_add_body,
        out_shape=jax.ShapeDtypeStruct((n, d), jnp.float32),
    )(partial)


# speedup vs baseline: 1.2718x; 1.0477x over previous
import functools

import jax
import jax.numpy as jnp
from jax import lax
from jax.experimental import pallas as pl
from jax.experimental.pallas import tpu as pltpu
from jax.experimental.pallas import tpu_sc as plsc

NC = 2
NS = 16
TILES = NC * NS
K = 128
BPT = 80
IB = 16
NB = BPT // IB


def _sc_partials(x, ei_pad, zrows, *, n_pad, d):
    stripe = n_pad // NS

    @functools.partial(
        pl.kernel,
        out_type=jax.ShapeDtypeStruct((NC, n_pad, d), jnp.float32),
        mesh=plsc.VectorSubcoreMesh(core_axis_name="c", subcore_axis_name="s"),
        scratch_types=[
            pltpu.VMEM((IB, K), jnp.int32),
            pltpu.VMEM((IB, K), jnp.int32),
            pltpu.VMEM((IB, K), jnp.int32),
            pltpu.VMEM((IB, K), jnp.int32),
            pltpu.VMEM((K, d), jnp.float32),
            pltpu.VMEM((K, d), jnp.float32),
            pltpu.VMEM_SHARED((n_pad, d), jnp.float32),
            pltpu.SemaphoreType.DMA,
            pltpu.SemaphoreType.DMA,
            pltpu.SemaphoreType.DMA,
            pltpu.SemaphoreType.DMA,
            pltpu.SemaphoreType.DMA,
            pltpu.SemaphoreType.DMA,
        ],
    )
    def sc_kernel(x_hbm, ei_hbm, zrows_hbm, out_hbm,
                  sidxA, didxA, sidxB, didxB, rows0, rows1, acc,
                  gs0, gs1, ss0, ss1, isemA, isemB):
        c = lax.axis_index("c")
        s = lax.axis_index("s")
        t = c * NS + s
        srcp_hbm = ei_hbm.at[0]
        dstp_hbm = ei_hbm.at[1]

        idxbufs = [(sidxA, didxA, isemA), (sidxB, didxB, isemB)]

        def start_idx(b, bufs):
            sidx, didx, isem = bufs
            base = t * BPT + b * IB
            pltpu.async_copy(srcp_hbm.at[pl.ds(base, IB)], sidx, isem)
            pltpu.async_copy(dstp_hbm.at[pl.ds(base, IB)], didx, isem)

        def wait_idx(bufs):
            sidx, didx, isem = bufs
            pltpu.make_async_copy(srcp_hbm.at[pl.ds(0, IB)], sidx, isem).wait()
            pltpu.make_async_copy(dstp_hbm.at[pl.ds(0, IB)], didx, isem).wait()

        def start_g(sidx, j, rows, sem):
            pltpu.async_copy(x_hbm.at[sidx.at[j]], rows, sem)

        def wait_g(sidx, j, rows, sem):
            pltpu.make_async_copy(x_hbm.at[sidx.at[j]], rows, sem).wait()

        def scat(didx, j, rows):
            pltpu.sync_copy(rows, acc.at[didx.at[j]], add=True)

        start_idx(0, idxbufs[0])
        wait_idx(idxbufs[0])
        start_idx(1, idxbufs[1])
        start_g(sidxA, 0, rows0, gs0)
        start_g(sidxA, 1, rows1, gs1)

        pltpu.sync_copy(zrows_hbm, acc.at[pl.ds(s * stripe, stripe)])
        plsc.subcore_barrier()

        for b in range(NB):
            sidx, didx, _ = idxbufs[b % 2]
            nxt = idxbufs[(b + 1) % 2]

            @pl.loop(0, IB, step=2)
            def _(j, sidx=sidx, didx=didx):
                wait_g(sidx, j, rows0, gs0)
                scat(didx, j, rows0)

                @pl.when(j + 2 < IB)
                def _():
                    start_g(sidx, j + 2, rows0, gs0)

                wait_g(sidx, j + 1, rows1, gs1)
                scat(didx, j + 1, rows1)

                @pl.when(j + 3 < IB)
                def _():
                    start_g(sidx, j + 3, rows1, gs1)

            if b + 1 < NB:
                wait_idx(nxt)
                start_g(nxt[0], 0, rows0, gs0)
                start_g(nxt[0], 1, rows1, gs1)
            if b + 2 < NB:
                start_idx(b + 2, idxbufs[b % 2])

        plsc.subcore_barrier()

        pltpu.sync_copy(acc.at[pl.ds(s * stripe, stripe)],
                        out_hbm.at[c].at[pl.ds(s * stripe, stripe)])

    return sc_kernel(x, ei_pad, zrows)


def _tc_add_body(p_ref, o_ref):
    n = o_ref.shape[0]
    o_ref[...] = p_ref[0, :n, :] + p_ref[1, :n, :]


def kernel(x, edge_index):
    n, d = x.shape
    e = edge_index.shape[1]
    n_pad = ((n + NS * 8 - 1) // (NS * 8)) * (NS * 8)
    e_pad = TILES * BPT * K
    pad = e_pad - e

    dummy_dst = n + jnp.arange(pad, dtype=jnp.int32) % (n_pad - n)
    dummy_src = jnp.arange(pad, dtype=jnp.int32) % n
    ei_pad = jnp.concatenate(
        [edge_index, jnp.stack([dummy_src, dummy_dst])], axis=1
    ).reshape(2, -1, K)
    zrows = jnp.zeros((n_pad // NS, d), jnp.float32)

    partial = _sc_partials(x, ei_pad, zrows, n_pad=n_pad, d=d)

    return pl.pallas_call(
        _tc_add_body,
        out_shape=jax.ShapeDtypeStruct((n, d), jnp.float32),
    )(partial)


# speedup vs baseline: 1.2839x; 1.0096x over previous
import functools

import jax
import jax.numpy as jnp
from jax import lax
from jax.experimental import pallas as pl
from jax.experimental.pallas import tpu as pltpu
from jax.experimental.pallas import tpu_sc as plsc

NC = 2
NS = 16
TILES = NC * NS
K = 128
IB = 16
AUX = 8


def _sc_partials(x, ei, aux, zrows, *, n_pad, d, bpt_main):
    stripe = n_pad // NS
    sizes = []
    off = 0
    while off < bpt_main:
        sizes.append(min(IB, bpt_main - off))
        off += sizes[-1]
    sizes.append(AUX)
    nblk = len(sizes)

    @functools.partial(
        pl.kernel,
        out_type=jax.ShapeDtypeStruct((NC, n_pad, d), jnp.float32),
        mesh=plsc.VectorSubcoreMesh(core_axis_name="c", subcore_axis_name="s"),
        scratch_types=[
            pltpu.VMEM((IB, K), jnp.int32),
            pltpu.VMEM((IB, K), jnp.int32),
            pltpu.VMEM((IB, K), jnp.int32),
            pltpu.VMEM((IB, K), jnp.int32),
            pltpu.VMEM((K, d), jnp.float32),
            pltpu.VMEM((K, d), jnp.float32),
            pltpu.VMEM_SHARED((n_pad, d), jnp.float32),
            pltpu.SemaphoreType.DMA,
            pltpu.SemaphoreType.DMA,
            pltpu.SemaphoreType.DMA,
            pltpu.SemaphoreType.DMA,
        ],
    )
    def sc_kernel(x_hbm, ei_hbm, aux_hbm, zrows_hbm, out_hbm,
                  sidxA, didxA, sidxB, didxB, rows0, rows1, acc,
                  gs0, gs1, isemA, isemB):
        c = lax.axis_index("c")
        s = lax.axis_index("s")
        t = c * NS + s

        idxbufs = [(sidxA, didxA, isemA), (sidxB, didxB, isemB)]
        blocks = [(ei_hbm.at[0], ei_hbm.at[1], t * bpt_main + boff, sz)
                  for boff, sz in zip([sum(sizes[:i]) for i in range(nblk - 1)],
                                      sizes[:-1])]
        blocks.append((aux_hbm.at[0], aux_hbm.at[1], t * AUX, AUX))

        def start_idx(b, bufs):
            sidx, didx, isem = bufs
            sref, dref, base, sz = blocks[b]
            pltpu.async_copy(sref.at[pl.ds(base, sz)],
                             sidx.at[pl.ds(0, sz)], isem)
            pltpu.async_copy(dref.at[pl.ds(base, sz)],
                             didx.at[pl.ds(0, sz)], isem)

        def wait_idx(b, bufs):
            sidx, didx, isem = bufs
            sref, dref, base, sz = blocks[b]
            pltpu.make_async_copy(sref.at[pl.ds(base, sz)],
                                  sidx.at[pl.ds(0, sz)], isem).wait()
            pltpu.make_async_copy(dref.at[pl.ds(base, sz)],
                                  didx.at[pl.ds(0, sz)], isem).wait()

        def start_g(sidx, j, rows, sem):
            pltpu.async_copy(x_hbm.at[sidx.at[j]], rows, sem)

        def wait_g(sidx, j, rows, sem):
            pltpu.make_async_copy(x_hbm.at[sidx.at[j]], rows, sem).wait()

        def scat(didx, j, rows):
            pltpu.sync_copy(rows, acc.at[didx.at[j]], add=True)

        start_idx(0, idxbufs[0])
        wait_idx(0, idxbufs[0])
        start_idx(1, idxbufs[1])
        start_g(sidxA, 0, rows0, gs0)
        start_g(sidxA, 1, rows1, gs1)

        pltpu.sync_copy(zrows_hbm, acc.at[pl.ds(s * stripe, stripe)])
        plsc.subcore_barrier()

        for b in range(nblk):
            sidx, didx, _ = idxbufs[b % 2]
            nxt = idxbufs[(b + 1) % 2]
            sz = blocks[b][3]

            @pl.loop(0, sz, step=2)
            def _(j, sidx=sidx, didx=didx, sz=sz):
                wait_g(sidx, j, rows0, gs0)
                scat(didx, j, rows0)

                @pl.when(j + 2 < sz)
                def _():
                    start_g(sidx, j + 2, rows0, gs0)

                wait_g(sidx, j + 1, rows1, gs1)
                scat(didx, j + 1, rows1)

                @pl.when(j + 3 < sz)
                def _():
                    start_g(sidx, j + 3, rows1, gs1)

            if b + 1 < nblk:
                wait_idx(b + 1, nxt)
                start_g(nxt[0], 0, rows0, gs0)
                start_g(nxt[0], 1, rows1, gs1)
            if b + 2 < nblk:
                start_idx(b + 2, idxbufs[b % 2])

        plsc.subcore_barrier()

        pltpu.sync_copy(acc.at[pl.ds(s * stripe, stripe)],
                        out_hbm.at[c].at[pl.ds(s * stripe, stripe)])

    return sc_kernel(x, ei, aux, zrows)


def _tc_add_body(p_ref, o_ref):
    n = o_ref.shape[0]
    o_ref[...] = p_ref[0, :n, :] + p_ref[1, :n, :]


def kernel(x, edge_index):
    n, d = x.shape
    e = edge_index.shape[1]
    n_pad = ((n + NS * 8 - 1) // (NS * 8)) * (NS * 8)

    nchunks = e // K
    bpt_main = (nchunks // TILES) // 8 * 8
    rem = nchunks - bpt_main * TILES
    naux = TILES * AUX
    pad = (naux - rem) * K

    ei = edge_index.reshape(2, nchunks, K)

    dummy_dst = n + jnp.arange(pad, dtype=jnp.int32) % (n_pad - n)
    dummy_src = jnp.arange(pad, dtype=jnp.int32) % n
    aux = jnp.concatenate(
        [ei[:, bpt_main * TILES:],
         jnp.stack([dummy_src, dummy_dst]).reshape(2, -1, K)], axis=1)
    zrows = jnp.zeros((n_pad // NS, d), jnp.float32)

    partial = _sc_partials(x, ei, aux, zrows, n_pad=n_pad, d=d,
                           bpt_main=bpt_main)

    return pl.pallas_call(
        _tc_add_body,
        out_shape=jax.ShapeDtypeStruct((n, d), jnp.float32),
    )(partial)


# speedup vs baseline: 1.3345x; 1.0394x over previous
import functools

import jax
import jax.numpy as jnp
from jax import lax
from jax.experimental import pallas as pl
from jax.experimental.pallas import tpu as pltpu
from jax.experimental.pallas import tpu_sc as plsc

NC = 2
NS = 16
TILES = NC * NS
K = 128
IB = 32
AUX = 8


def _sc_partials(x, ei, aux, zrows, *, n_pad, d, bpt_main):
    stripe = n_pad // NS
    sizes = []
    off = 0
    while off < bpt_main:
        sizes.append(min(IB, bpt_main - off))
        off += sizes[-1]
    sizes.append(AUX)
    nblk = len(sizes)

    @functools.partial(
        pl.kernel,
        out_type=jax.ShapeDtypeStruct((NC, n_pad, d), jnp.float32),
        mesh=plsc.VectorSubcoreMesh(core_axis_name="c", subcore_axis_name="s"),
        scratch_types=[
            pltpu.VMEM((IB, K), jnp.int32),
            pltpu.VMEM((IB, K), jnp.int32),
            pltpu.VMEM((IB, K), jnp.int32),
            pltpu.VMEM((IB, K), jnp.int32),
            pltpu.VMEM((K, d), jnp.float32),
            pltpu.VMEM((K, d), jnp.float32),
            pltpu.VMEM_SHARED((n_pad, d), jnp.float32),
            pltpu.SemaphoreType.DMA,
            pltpu.SemaphoreType.DMA,
            pltpu.SemaphoreType.DMA,
            pltpu.SemaphoreType.DMA,
        ],
    )
    def sc_kernel(x_hbm, ei_hbm, aux_hbm, zrows_hbm, out_hbm,
                  sidxA, didxA, sidxB, didxB, rows0, rows1, acc,
                  gs0, gs1, isemA, isemB):
        c = lax.axis_index("c")
        s = lax.axis_index("s")
        t = c * NS + s

        idxbufs = [(sidxA, didxA, isemA), (sidxB, didxB, isemB)]
        blocks = [(ei_hbm.at[0], ei_hbm.at[1], t * bpt_main + boff, sz)
                  for boff, sz in zip([sum(sizes[:i]) for i in range(nblk - 1)],
                                      sizes[:-1])]
        blocks.append((aux_hbm.at[0], aux_hbm.at[1], t * AUX, AUX))

        def start_idx(b, bufs):
            sidx, didx, isem = bufs
            sref, dref, base, sz = blocks[b]
            pltpu.async_copy(sref.at[pl.ds(base, sz)],
                             sidx.at[pl.ds(0, sz)], isem)
            pltpu.async_copy(dref.at[pl.ds(base, sz)],
                             didx.at[pl.ds(0, sz)], isem)

        def wait_idx(b, bufs):
            sidx, didx, isem = bufs
            sref, dref, base, sz = blocks[b]
            pltpu.make_async_copy(sref.at[pl.ds(base, sz)],
                                  sidx.at[pl.ds(0, sz)], isem).wait()
            pltpu.make_async_copy(dref.at[pl.ds(base, sz)],
                                  didx.at[pl.ds(0, sz)], isem).wait()

        def start_g(sidx, j, rows, sem):
            pltpu.async_copy(x_hbm.at[sidx.at[j]], rows, sem)

        def wait_g(sidx, j, rows, sem):
            pltpu.make_async_copy(x_hbm.at[sidx.at[j]], rows, sem).wait()

        def scat(didx, j, rows):
            pltpu.sync_copy(rows, acc.at[didx.at[j]], add=True)

        start_idx(0, idxbufs[0])
        wait_idx(0, idxbufs[0])
        start_idx(1, idxbufs[1])
        start_g(sidxA, 0, rows0, gs0)
        start_g(sidxA, 1, rows1, gs1)

        pltpu.sync_copy(zrows_hbm, acc.at[pl.ds(s * stripe, stripe)])
        plsc.subcore_barrier()

        for b in range(nblk):
            sidx, didx, _ = idxbufs[b % 2]
            nxt = idxbufs[(b + 1) % 2]
            sz = blocks[b][3]

            @pl.loop(0, sz, step=2)
            def _(j, sidx=sidx, didx=didx, sz=sz):
                wait_g(sidx, j, rows0, gs0)
                scat(didx, j, rows0)

                @pl.when(j + 2 < sz)
                def _():
                    start_g(sidx, j + 2, rows0, gs0)

                wait_g(sidx, j + 1, rows1, gs1)
                scat(didx, j + 1, rows1)

                @pl.when(j + 3 < sz)
                def _():
                    start_g(sidx, j + 3, rows1, gs1)

            if b + 1 < nblk:
                wait_idx(b + 1, nxt)
                start_g(nxt[0], 0, rows0, gs0)
                start_g(nxt[0], 1, rows1, gs1)
            if b + 2 < nblk:
                start_idx(b + 2, idxbufs[b % 2])

        plsc.subcore_barrier()

        pltpu.sync_copy(acc.at[pl.ds(s * stripe, stripe)],
                        out_hbm.at[c].at[pl.ds(s * stripe, stripe)])

    return sc_kernel(x, ei, aux, zrows)


def _tc_add_body(p_ref, o_ref):
    n = o_ref.shape[0]
    o_ref[...] = p_ref[0, :n, :] + p_ref[1, :n, :]


def kernel(x, edge_index):
    n, d = x.shape
    e = edge_index.shape[1]
    n_pad = ((n + NS * 8 - 1) // (NS * 8)) * (NS * 8)

    nchunks = e // K
    bpt_main = (nchunks // TILES) // 8 * 8
    rem = nchunks - bpt_main * TILES
    naux = TILES * AUX
    pad = (naux - rem) * K

    ei = edge_index.reshape(2, nchunks, K)

    dummy_dst = n + jnp.arange(pad, dtype=jnp.int32) % (n_pad - n)
    dummy_src = jnp.arange(pad, dtype=jnp.int32) % n
    aux = jnp.concatenate(
        [ei[:, bpt_main * TILES:],
         jnp.stack([dummy_src, dummy_dst]).reshape(2, -1, K)], axis=1)
    zrows = jnp.zeros((n_pad // NS, d), jnp.float32)

    partial = _sc_partials(x, ei, aux, zrows, n_pad=n_pad, d=d,
                           bpt_main=bpt_main)

    return pl.pallas_call(
        _tc_add_body,
        out_shape=jax.ShapeDtypeStruct((n, d), jnp.float32),
    )(partial)
